# baseline (speedup 1.0000x reference)
import functools

import jax
import jax.numpy as jnp
import numpy as np
from jax.experimental import pallas as pl
import jax.experimental.pallas.tpu as pltpu

N = 50000
SCALAR_DIM = 512
VECTOR_DIM = 128
HID = 128
OUT_DIM = 128
NUM_GRAPHS = 64
TILE = 5000
NUM_TILES = N // TILE


def _leaky(x):
    return jnp.where(x >= 0, x, 0.01 * x)


def _fused_kernel(scalar_ref, vec_ref, batch_ref, w_ref, bg1_ref,
                  wg2_ref, bg2_ref, bm1_ref, wm2_ref, bm2_ref, out_ref,
                  acc_ref, m_ref, s_ref):
    i = pl.program_id(0)

    @pl.when(i == 0)
    def _init():
        acc_ref[...] = jnp.zeros_like(acc_ref)
        m_ref[...] = jnp.full_like(m_ref, -1e30)
        s_ref[...] = jnp.zeros_like(s_ref)

    y = jnp.dot(scalar_ref[...].astype(jnp.bfloat16), w_ref[:SCALAR_DIM, :],
                preferred_element_type=jnp.float32)
    for k in range(3):
        wk = w_ref[SCALAR_DIM + k * VECTOR_DIM:SCALAR_DIM + (k + 1) * VECTOR_DIM, :]
        y += jnp.dot(vec_ref[k].astype(jnp.bfloat16), wk,
                     preferred_element_type=jnp.float32)
    y_g = y[:, :HID]
    y_m = y[:, HID:]

    h = _leaky(y_g + bg1_ref[...])
    gate = jnp.sum(h * wg2_ref[...], axis=1, keepdims=True) + bg2_ref[0, 0]

    batch_t = batch_ref[0, 0, :]
    seg_ids = jax.lax.broadcasted_iota(jnp.int32, (TILE, NUM_GRAPHS), 1)
    onehot_b = batch_t[:, None] == seg_ids
    onehot = onehot_b.astype(jnp.float32)

    tile_max = jnp.max(jnp.where(onehot_b, gate, -1e30), axis=0)
    m_old = m_ref[0, :]
    m_new = jnp.maximum(m_old, tile_max)
    scale = jnp.exp(m_old - m_new)

    m_per_node = jnp.dot(onehot, m_new, preferred_element_type=jnp.float32)
    e = jnp.exp(gate[:, 0] - m_per_node)

    s_new = s_ref[0, :] * scale + jnp.sum(onehot * e[:, None], axis=0)
    acc_new = acc_ref[...] * scale[:, None] + jax.lax.dot_general(
        onehot, e[:, None] * y_m,
        dimension_numbers=(((0,), (0,)), ((), ())),
        preferred_element_type=jnp.float32)

    m_ref[0, :] = m_new
    s_ref[0, :] = s_new
    acc_ref[...] = acc_new

    @pl.when(i == NUM_TILES - 1)
    def _finish():
        seg = acc_new / (s_new[:, None] + 1e-16)
        o1 = _leaky(seg + bm1_ref[...])
        out_ref[...] = jnp.dot(o1, wm2_ref[...],
                               preferred_element_type=jnp.float32) + bm2_ref[...]


@functools.partial(jax.jit, static_argnames=())
def kernel(scalar, vector, batch, W_g1, b_g1, W_g2, b_g2, W_m1, b_m1, W_m2,
           b_m2):
    vec3 = vector.transpose(2, 0, 1)
    batch3d = batch.astype(jnp.int32).reshape(NUM_TILES, 1, TILE)
    perm = np.concatenate([
        np.arange(SCALAR_DIM),
        SCALAR_DIM + (np.arange(3)[:, None] +
                      3 * np.arange(VECTOR_DIM)[None, :]).reshape(-1),
    ])
    w_cat = jnp.concatenate([W_g1, W_m1], axis=1)[perm].astype(jnp.bfloat16)

    grid = (NUM_TILES,)
    out = pl.pallas_call(
        _fused_kernel,
        grid=grid,
        in_specs=[
            pl.BlockSpec((TILE, SCALAR_DIM), lambda i: (i, 0)),
            pl.BlockSpec((3, TILE, VECTOR_DIM), lambda i: (0, i, 0)),
            pl.BlockSpec((1, 1, TILE), lambda i: (i, 0, 0)),
            pl.BlockSpec((SCALAR_DIM + 3 * VECTOR_DIM, 2 * HID),
                         lambda i: (0, 0)),
            pl.BlockSpec((1, HID), lambda i: (0, 0)),
            pl.BlockSpec((1, HID), lambda i: (0, 0)),
            pl.BlockSpec((1, 1), lambda i: (0, 0)),
            pl.BlockSpec((1, OUT_DIM), lambda i: (0, 0)),
            pl.BlockSpec((OUT_DIM, OUT_DIM), lambda i: (0, 0)),
            pl.BlockSpec((1, OUT_DIM), lambda i: (0, 0)),
        ],
        out_specs=pl.BlockSpec((NUM_GRAPHS, OUT_DIM), lambda i: (0, 0)),
        out_shape=jax.ShapeDtypeStruct((NUM_GRAPHS, OUT_DIM), jnp.float32),
        scratch_shapes=[
            pltpu.VMEM((NUM_GRAPHS, OUT_DIM), jnp.float32),
            pltpu.VMEM((1, NUM_GRAPHS), jnp.float32),
            pltpu.VMEM((1, NUM_GRAPHS), jnp.float32),
        ],
    )(scalar, vec3, batch3d, w_cat, b_g1.reshape(1, HID),
      W_g2.reshape(1, HID), b_g2.reshape(1, 1), b_m1.reshape(1, OUT_DIM),
      W_m2, b_m2.reshape(1, OUT_DIM))
    return out


# speedup vs baseline: 1.0731x; 1.0731x over previous
import functools

import jax
import jax.numpy as jnp
from jax.experimental import pallas as pl
import jax.experimental.pallas.tpu as pltpu

N = 50000
SCALAR_DIM = 512
VECTOR_DIM = 128
INPUT_DIM = SCALAR_DIM + 3 * VECTOR_DIM
HID = 128
OUT_DIM = 128
NUM_GRAPHS = 64
TILE = 5000
NUM_TILES = N // TILE


def _leaky(x):
    return jnp.where(x >= 0, x, 0.01 * x)


def _fused_kernel(scalar_ref, vec_ref, batch_ref, wg1_ref, wm1_ref, bg1_ref,
                  wg2_ref, bg2_ref, bm1_ref, wm2_ref, bm2_ref, out_ref,
                  acc_ref, m_ref, s_ref, w_ref):
    i = pl.program_id(0)

    @pl.when(i == 0)
    def _init():
        acc_ref[...] = jnp.zeros_like(acc_ref)
        m_ref[...] = jnp.full_like(m_ref, -1e30)
        s_ref[...] = jnp.zeros_like(s_ref)
        w_cat = jnp.concatenate([wg1_ref[...], wm1_ref[...]],
                                axis=1).astype(jnp.bfloat16)
        w_ref[:SCALAR_DIM, :] = w_cat[:SCALAR_DIM, :]
        row = jax.lax.broadcasted_iota(jnp.int32, (3 * VECTOR_DIM,
                                                   3 * VECTOR_DIM), 0)
        col = jax.lax.broadcasted_iota(jnp.int32, (3 * VECTOR_DIM,
                                                   3 * VECTOR_DIM), 1)
        sel = (col == 3 * (row % VECTOR_DIM) + row // VECTOR_DIM)
        w_ref[SCALAR_DIM:, :] = jnp.dot(
            sel.astype(jnp.bfloat16), w_cat[SCALAR_DIM:, :],
            preferred_element_type=jnp.float32).astype(jnp.bfloat16)

    y = jnp.dot(scalar_ref[...].astype(jnp.bfloat16), w_ref[:SCALAR_DIM, :],
                preferred_element_type=jnp.float32)
    for k in range(3):
        wk = w_ref[SCALAR_DIM + k * VECTOR_DIM:
                   SCALAR_DIM + (k + 1) * VECTOR_DIM, :]
        y += jnp.dot(vec_ref[k].astype(jnp.bfloat16), wk,
                     preferred_element_type=jnp.float32)
    y_g = y[:, :HID]
    y_m = y[:, HID:]

    h = _leaky(y_g + bg1_ref[...])
    gate = jnp.sum(h * wg2_ref[...], axis=1, keepdims=True) + bg2_ref[0, 0]

    batch_t = batch_ref[0, 0, :]
    seg_ids = jax.lax.broadcasted_iota(jnp.int32, (TILE, NUM_GRAPHS), 1)
    onehot_b = batch_t[:, None] == seg_ids
    onehot = onehot_b.astype(jnp.float32)

    tile_max = jnp.max(jnp.where(onehot_b, gate, -1e30), axis=0)
    m_old = m_ref[0, :]
    m_new = jnp.maximum(m_old, tile_max)
    scale = jnp.exp(m_old - m_new)

    m_per_node = jnp.dot(onehot, m_new, preferred_element_type=jnp.float32)
    e = jnp.exp(gate[:, 0] - m_per_node)

    s_new = s_ref[0, :] * scale + jnp.sum(onehot * e[:, None], axis=0)
    acc_new = acc_ref[...] * scale[:, None] + jax.lax.dot_general(
        onehot, e[:, None] * y_m,
        dimension_numbers=(((0,), (0,)), ((), ())),
        preferred_element_type=jnp.float32)

    m_ref[0, :] = m_new
    s_ref[0, :] = s_new
    acc_ref[...] = acc_new

    @pl.when(i == NUM_TILES - 1)
    def _finish():
        seg = acc_new / (s_new[:, None] + 1e-16)
        o1 = _leaky(seg + bm1_ref[...])
        out_ref[...] = jnp.dot(o1, wm2_ref[...],
                               preferred_element_type=jnp.float32) + bm2_ref[...]


@functools.partial(jax.jit, static_argnames=())
def kernel(scalar, vector, batch, W_g1, b_g1, W_g2, b_g2, W_m1, b_m1, W_m2,
           b_m2):
    vec3 = vector.transpose(2, 0, 1)
    batch3d = batch.astype(jnp.int32).reshape(NUM_TILES, 1, TILE)

    grid = (NUM_TILES,)
    out = pl.pallas_call(
        _fused_kernel,
        grid=grid,
        in_specs=[
            pl.BlockSpec((TILE, SCALAR_DIM), lambda i: (i, 0)),
            pl.BlockSpec((3, TILE, VECTOR_DIM), lambda i: (0, i, 0)),
            pl.BlockSpec((1, 1, TILE), lambda i: (i, 0, 0)),
            pl.BlockSpec((INPUT_DIM, HID), lambda i: (0, 0)),
            pl.BlockSpec((INPUT_DIM, OUT_DIM), lambda i: (0, 0)),
            pl.BlockSpec((1, HID), lambda i: (0, 0)),
            pl.BlockSpec((1, HID), lambda i: (0, 0)),
            pl.BlockSpec((1, 1), lambda i: (0, 0)),
            pl.BlockSpec((1, OUT_DIM), lambda i: (0, 0)),
            pl.BlockSpec((OUT_DIM, OUT_DIM), lambda i: (0, 0)),
            pl.BlockSpec((1, OUT_DIM), lambda i: (0, 0)),
        ],
        out_specs=pl.BlockSpec((NUM_GRAPHS, OUT_DIM), lambda i: (0, 0)),
        out_shape=jax.ShapeDtypeStruct((NUM_GRAPHS, OUT_DIM), jnp.float32),
        scratch_shapes=[
            pltpu.VMEM((NUM_GRAPHS, OUT_DIM), jnp.float32),
            pltpu.VMEM((1, NUM_GRAPHS), jnp.float32),
            pltpu.VMEM((1, NUM_GRAPHS), jnp.float32),
            pltpu.VMEM((INPUT_DIM, 2 * HID), jnp.bfloat16),
        ],
    )(scalar, vec3, batch3d, W_g1, W_m1, b_g1.reshape(1, HID),
      W_g2.reshape(1, HID), b_g2.reshape(1, 1), b_m1.reshape(1, OUT_DIM),
      W_m2, b_m2.reshape(1, OUT_DIM))
    return out
